# baseline (speedup 1.0000x reference)
import functools

import jax
import jax.numpy as jnp
from jax.experimental import pallas as pl

N_ROWS = 4096
N_COLS = 200
BLOCK_ROWS = 8


def _listmle_block_kernel(preds_ref, targets_ref, out_ref):
    t = targets_ref[:]
    p = preds_ref[:]
    m = jnp.max(p, axis=-1, keepdims=True)
    sh = p - m
    e = jnp.exp(sh)

    ti = t[:, None, :]
    tj = t[:, :, None]
    ii = jax.lax.broadcasted_iota(jnp.int32, (1, N_COLS, N_COLS), 2)
    jj = jax.lax.broadcasted_iota(jnp.int32, (1, N_COLS, N_COLS), 1)
    cmp = (tj < ti) | ((tj == ti) & (jj >= ii))

    s = jnp.sum(jnp.where(cmp, e[:, :, None], 0.0), axis=1)
    nll = jnp.sum(jnp.log(s) - sh, axis=-1)

    @pl.when(pl.program_id(0) == 0)
    def _init():
        out_ref[:, :] = jnp.zeros((1, 1), jnp.float32)

    out_ref[:, :] += jnp.sum(nll).reshape(1, 1)


@jax.jit
def kernel(preds, targets):
    grid = (N_ROWS // BLOCK_ROWS,)
    total = pl.pallas_call(
        _listmle_block_kernel,
        grid=grid,
        in_specs=[
            pl.BlockSpec((BLOCK_ROWS, N_COLS), lambda b: (b, 0)),
            pl.BlockSpec((BLOCK_ROWS, N_COLS), lambda b: (b, 0)),
        ],
        out_specs=pl.BlockSpec((1, 1), lambda b: (0, 0)),
        out_shape=jax.ShapeDtypeStruct((1, 1), jnp.float32),
    )(preds, targets)
    return total[0, 0] / N_ROWS


# speedup vs baseline: 1.3496x; 1.3496x over previous
import functools

import jax
import jax.numpy as jnp
from jax.experimental import pallas as pl

N_ROWS = 4096
N_COLS = 200
BLOCK_ROWS = 32


def _listmle_block_kernel(preds_ref, targets_ref, out_ref):
    t = targets_ref[:]
    p = preds_ref[:]
    m = jnp.max(p, axis=-1, keepdims=True)
    sh = p - m
    e = jnp.exp(sh)

    ti = t[:, None, :]
    tj = t[:, :, None]
    ii = jax.lax.broadcasted_iota(jnp.int32, (1, N_COLS, N_COLS), 2)
    jj = jax.lax.broadcasted_iota(jnp.int32, (1, N_COLS, N_COLS), 1)
    cmp = (tj < ti) | ((tj == ti) & (jj >= ii))

    s = jnp.sum(jnp.where(cmp, e[:, :, None], 0.0), axis=1)
    nll = jnp.sum(jnp.log(s) - sh, axis=-1)

    @pl.when(pl.program_id(0) == 0)
    def _init():
        out_ref[:, :] = jnp.zeros((1, 1), jnp.float32)

    out_ref[:, :] += jnp.sum(nll).reshape(1, 1)


@jax.jit
def kernel(preds, targets):
    grid = (N_ROWS // BLOCK_ROWS,)
    total = pl.pallas_call(
        _listmle_block_kernel,
        grid=grid,
        in_specs=[
            pl.BlockSpec((BLOCK_ROWS, N_COLS), lambda b: (b, 0)),
            pl.BlockSpec((BLOCK_ROWS, N_COLS), lambda b: (b, 0)),
        ],
        out_specs=pl.BlockSpec((1, 1), lambda b: (0, 0)),
        out_shape=jax.ShapeDtypeStruct((1, 1), jnp.float32),
    )(preds, targets)
    return total[0, 0] / N_ROWS


# speedup vs baseline: 1.5011x; 1.1122x over previous
import functools

import jax
import jax.numpy as jnp
from jax.experimental import pallas as pl

N_ROWS = 4096
N_COLS = 200
BLOCK_ROWS = 32


def _listmle_block_kernel(preds_ref, targets_ref, out_ref):
    t = targets_ref[:]
    p = preds_ref[:]
    m = jnp.max(p, axis=-1, keepdims=True)
    sh = p - m
    e = jnp.exp(sh)

    b = jax.lax.bitcast_convert_type(t, jnp.int32)
    k = b ^ ((b >> 31) & jnp.int32(0x7FFFFFFF))

    ki = k[:, None, :]
    kj = k[:, :, None]
    ii = jax.lax.broadcasted_iota(jnp.int32, (1, N_COLS, N_COLS), 2)
    jj = jax.lax.broadcasted_iota(jnp.int32, (1, N_COLS, N_COLS), 1)
    tie = (jj >= ii).astype(jnp.int32)

    cmp = kj < (ki + tie)
    s = jnp.sum(jnp.where(cmp, e[:, :, None], 0.0), axis=1)
    nll = jnp.sum(jnp.log(s) - sh, axis=-1)

    @pl.when(pl.program_id(0) == 0)
    def _init():
        out_ref[:, :] = jnp.zeros((1, 1), jnp.float32)

    out_ref[:, :] += jnp.sum(nll).reshape(1, 1)


@jax.jit
def kernel(preds, targets):
    grid = (N_ROWS // BLOCK_ROWS,)
    total = pl.pallas_call(
        _listmle_block_kernel,
        grid=grid,
        in_specs=[
            pl.BlockSpec((BLOCK_ROWS, N_COLS), lambda b: (b, 0)),
            pl.BlockSpec((BLOCK_ROWS, N_COLS), lambda b: (b, 0)),
        ],
        out_specs=pl.BlockSpec((1, 1), lambda b: (0, 0)),
        out_shape=jax.ShapeDtypeStruct((1, 1), jnp.float32),
    )(preds, targets)
    return total[0, 0] / N_ROWS


# speedup vs baseline: 1.5983x; 1.0648x over previous
import functools

import jax
import jax.numpy as jnp
from jax.experimental import pallas as pl

N_ROWS = 4096
N_COLS = 200
BLOCK_ROWS = 64


def _listmle_block_kernel(preds_ref, targets_ref, out_ref):
    t = targets_ref[:]
    p = preds_ref[:]
    m = jnp.max(p, axis=-1, keepdims=True)
    sh = p - m
    e = jnp.exp(sh)

    b = jax.lax.bitcast_convert_type(t, jnp.int32)
    k = b ^ ((b >> 31) & jnp.int32(0x7FFFFFFF))

    ki = k[:, None, :]
    kj = k[:, :, None]
    ii = jax.lax.broadcasted_iota(jnp.int32, (1, N_COLS, N_COLS), 2)
    jj = jax.lax.broadcasted_iota(jnp.int32, (1, N_COLS, N_COLS), 1)
    tie = (jj >= ii).astype(jnp.int32)

    cmp = kj < (ki + tie)
    s = jnp.sum(jnp.where(cmp, e[:, :, None], 0.0), axis=1)
    nll = jnp.sum(jnp.log(s) - sh, axis=-1)

    @pl.when(pl.program_id(0) == 0)
    def _init():
        out_ref[:, :] = jnp.zeros((1, 1), jnp.float32)

    out_ref[:, :] += jnp.sum(nll).reshape(1, 1)


@jax.jit
def kernel(preds, targets):
    grid = (N_ROWS // BLOCK_ROWS,)
    total = pl.pallas_call(
        _listmle_block_kernel,
        grid=grid,
        in_specs=[
            pl.BlockSpec((BLOCK_ROWS, N_COLS), lambda b: (b, 0)),
            pl.BlockSpec((BLOCK_ROWS, N_COLS), lambda b: (b, 0)),
        ],
        out_specs=pl.BlockSpec((1, 1), lambda b: (0, 0)),
        out_shape=jax.ShapeDtypeStruct((1, 1), jnp.float32),
    )(preds, targets)
    return total[0, 0] / N_ROWS


# speedup vs baseline: 1.6458x; 1.0297x over previous
import functools

import jax
import jax.numpy as jnp
from jax.experimental import pallas as pl

N_ROWS = 4096
N_COLS = 200
BLOCK_ROWS = 128


def _listmle_block_kernel(preds_ref, targets_ref, out_ref):
    t = targets_ref[:]
    p = preds_ref[:]
    m = jnp.max(p, axis=-1, keepdims=True)
    sh = p - m
    e = jnp.exp(sh)

    b = jax.lax.bitcast_convert_type(t, jnp.int32)
    k = b ^ ((b >> 31) & jnp.int32(0x7FFFFFFF))

    ki = k[:, None, :]
    kj = k[:, :, None]
    ii = jax.lax.broadcasted_iota(jnp.int32, (1, N_COLS, N_COLS), 2)
    jj = jax.lax.broadcasted_iota(jnp.int32, (1, N_COLS, N_COLS), 1)
    tie = (jj >= ii).astype(jnp.int32)

    cmp = kj < (ki + tie)
    s = jnp.sum(jnp.where(cmp, e[:, :, None], 0.0), axis=1)
    nll = jnp.sum(jnp.log(s) - sh, axis=-1)

    @pl.when(pl.program_id(0) == 0)
    def _init():
        out_ref[:, :] = jnp.zeros((1, 1), jnp.float32)

    out_ref[:, :] += jnp.sum(nll).reshape(1, 1)


@jax.jit
def kernel(preds, targets):
    grid = (N_ROWS // BLOCK_ROWS,)
    total = pl.pallas_call(
        _listmle_block_kernel,
        grid=grid,
        in_specs=[
            pl.BlockSpec((BLOCK_ROWS, N_COLS), lambda b: (b, 0)),
            pl.BlockSpec((BLOCK_ROWS, N_COLS), lambda b: (b, 0)),
        ],
        out_specs=pl.BlockSpec((1, 1), lambda b: (0, 0)),
        out_shape=jax.ShapeDtypeStruct((1, 1), jnp.float32),
    )(preds, targets)
    return total[0, 0] / N_ROWS


# speedup vs baseline: 2.4512x; 1.4894x over previous
import jax
import jax.numpy as jnp
from jax.experimental import pallas as pl

N_ROWS = 4096
N_COLS = 200
LANES = 128
IC = 40


def _f32_sort_key(x):
    b = jax.lax.bitcast_convert_type(x, jnp.int32)
    return b ^ ((b >> 31) & jnp.int32(0x7FFFFFFF))


def _listmle_block_kernel(tT_j_ref, pT_j_ref, tT_i_ref, out_ref):
    ic = pl.program_id(1)

    tTj = tT_j_ref[:]
    pTj = pT_j_ref[:]
    kTj = _f32_sort_key(tTj)
    kTi = _f32_sort_key(tT_i_ref[:])

    m = jnp.max(pTj, axis=0, keepdims=True)
    shT = pTj - m
    eT = jnp.exp(shT)

    ki3 = kTi[:, None, :]
    kj3 = kTj[None, :, :]
    ii = jax.lax.broadcasted_iota(jnp.int32, (IC, N_COLS, 1), 0) + ic * IC
    jj = jax.lax.broadcasted_iota(jnp.int32, (IC, N_COLS, 1), 1)
    tie = (jj >= ii).astype(jnp.int32)

    cmp = kj3 < (ki3 + tie)
    s = jnp.sum(jnp.where(cmp, eT[None, :, :], 0.0), axis=1)
    part = jnp.sum(jnp.log(s), axis=0, keepdims=True)

    @pl.when((pl.program_id(0) == 0) & (ic == 0))
    def _init():
        out_ref[:, :] = jnp.zeros((1, LANES), jnp.float32)

    @pl.when(ic == 0)
    def _sub_shifted():
        out_ref[:, :] -= jnp.sum(shT, axis=0, keepdims=True)

    out_ref[:, :] += part


@jax.jit
def kernel(preds, targets):
    pT = preds.T
    tT = targets.T
    grid = (N_ROWS // LANES, N_COLS // IC)
    acc = pl.pallas_call(
        _listmle_block_kernel,
        grid=grid,
        in_specs=[
            pl.BlockSpec((N_COLS, LANES), lambda b, ic: (0, b)),
            pl.BlockSpec((N_COLS, LANES), lambda b, ic: (0, b)),
            pl.BlockSpec((IC, LANES), lambda b, ic: (ic, b)),
        ],
        out_specs=pl.BlockSpec((1, LANES), lambda b, ic: (0, 0)),
        out_shape=jax.ShapeDtypeStruct((1, LANES), jnp.float32),
    )(tT, pT, tT)
    return jnp.sum(acc) / N_ROWS


# speedup vs baseline: 4.7045x; 1.9193x over previous
import jax
import jax.numpy as jnp
from jax.experimental import pallas as pl

N_ROWS = 4096
N_COLS = 200
LANES = 128
IC = 40


def _f32_sort_key(x):
    b = jax.lax.bitcast_convert_type(x, jnp.int32)
    return b ^ ((b >> 31) & jnp.int32(0x7FFFFFFF))


def _listmle_block_kernel(tT_ref, pT_ref, out_ref):
    tT = tT_ref[:]
    pT = pT_ref[:]
    kT = _f32_sort_key(tT)

    m = jnp.max(pT, axis=0, keepdims=True)
    shT = pT - m
    eT = jnp.exp(shT)
    e3 = eT[None, :, :]
    k3 = kT[None, :, :]

    acc = jnp.zeros((1, LANES), jnp.float32) - jnp.sum(shT, axis=0,
                                                       keepdims=True)
    for ci in range(N_COLS // IC):
        i0 = ci * IC
        i1 = i0 + IC
        ki3 = kT[i0:i1][:, None, :]
        ki3t = ki3 + 1
        s = jnp.zeros((IC, LANES), jnp.float32)
        if i0 > 0:
            s += jnp.sum(jnp.where(k3[:, :i0] < ki3, e3[:, :i0], 0.0),
                         axis=1)
        if i1 < N_COLS:
            s += jnp.sum(jnp.where(k3[:, i1:] < ki3t, e3[:, i1:], 0.0),
                         axis=1)
        ii = jax.lax.broadcasted_iota(jnp.int32, (IC, IC, 1), 0)
        jj = jax.lax.broadcasted_iota(jnp.int32, (IC, IC, 1), 1)
        kiL = jnp.where(jj >= ii, ki3t, ki3)
        s += jnp.sum(jnp.where(k3[:, i0:i1] < kiL, e3[:, i0:i1], 0.0),
                     axis=1)
        acc += jnp.sum(jnp.log(s), axis=0, keepdims=True)

    @pl.when(pl.program_id(0) == 0)
    def _init():
        out_ref[:, :] = jnp.zeros((1, LANES), jnp.float32)

    out_ref[:, :] += acc


@jax.jit
def kernel(preds, targets):
    pT = preds.T
    tT = targets.T
    grid = (N_ROWS // LANES,)
    acc = pl.pallas_call(
        _listmle_block_kernel,
        grid=grid,
        in_specs=[
            pl.BlockSpec((N_COLS, LANES), lambda b: (0, b)),
            pl.BlockSpec((N_COLS, LANES), lambda b: (0, b)),
        ],
        out_specs=pl.BlockSpec((1, LANES), lambda b: (0, 0)),
        out_shape=jax.ShapeDtypeStruct((1, LANES), jnp.float32),
    )(tT, pT)
    return jnp.sum(acc) / N_ROWS
